# baseline (speedup 1.0000x reference)
import functools

import jax
import jax.numpy as jnp
from jax import lax
from jax.experimental import pallas as pl
from jax.experimental.pallas import tpu as pltpu
from jax.experimental.pallas import tpu_sc as plsc

_NUM_FIELDS = 26
_PER_FIELD_VOCAB = 100000
_EMBED_DIM = 32
_BATCH = 16384

_NC = 2
_NS = 16
_L = 16
_NW = _NC * _NS
_BW = _BATCH // _NW
_QROWS = 650000

_mesh = plsc.VectorSubcoreMesh(core_axis_name="c", subcore_axis_name="s")


@functools.partial(
    pl.kernel,
    out_type=jax.ShapeDtypeStruct((_NUM_FIELDS, _EMBED_DIM, _BATCH), jnp.float32),
    mesh=_mesh,
    compiler_params=pltpu.CompilerParams(needs_layout_passes=False),
    scratch_types=[
        pltpu.VMEM((_NUM_FIELDS, _BW), jnp.int32),
        pltpu.VMEM((4, 128), jnp.int32),
        pltpu.VMEM((_BW,), jnp.int32),
        pltpu.VMEM((_BW, 128), jnp.float32),
        pltpu.VMEM((_EMBED_DIM, _BW), jnp.float32),
        pltpu.SemaphoreType.DMA,
    ],
)
def _emb_lookup(xT_hbm, tq_hbm, out_hbm, xb_v, qidx_v, sub_v, rows_v, out_v, sem):
    wid = lax.axis_index("s") * _NC + lax.axis_index("c")
    b0 = wid * _BW
    pltpu.sync_copy(xT_hbm.at[:, pl.ds(b0, _BW)], xb_v)

    def field_body(f, carry):
        off = f * _PER_FIELD_VOCAB

        def idx_grp(g, c):
            v = xb_v[f, pl.ds(g * _L, _L)] + off
            qidx_v[g >> 3, pl.ds((g & 7) * _L, _L)] = lax.shift_right_logical(v, 2)
            sub_v[pl.ds(g * _L, _L)] = lax.shift_left(jnp.bitwise_and(v, 3), 5)
            return c

        lax.fori_loop(0, _BW // _L, idx_grp, 0)

        copies = [
            pltpu.async_copy(
                tq_hbm.at[qidx_v.at[k]], rows_v.at[pl.ds(k * 128, 128)], sem
            )
            for k in range(4)
        ]
        for c in copies:
            c.wait()

        def ext_grp(g, c):
            jvec = g * _L + lax.iota(jnp.int32, _L)
            svec = sub_v[pl.ds(g * _L, _L)]
            for e in range(_EMBED_DIM):
                out_v[e, pl.ds(g * _L, _L)] = plsc.load_gather(
                    rows_v, [jvec, svec + e]
                )
            return c

        lax.fori_loop(0, _BW // _L, ext_grp, 0)
        pltpu.sync_copy(out_v, out_hbm.at[f, :, pl.ds(b0, _BW)])
        return carry

    lax.fori_loop(0, _NUM_FIELDS, field_body, 0)


@jax.jit
def kernel(x, embedding_table):
    out = _emb_lookup(x.T, embedding_table.reshape(_QROWS, 128))
    return jnp.transpose(out, (2, 0, 1))


# speedup vs baseline: 1.1840x; 1.1840x over previous
import functools

import jax
import jax.numpy as jnp
from jax import lax
from jax.experimental import pallas as pl
from jax.experimental.pallas import tpu as pltpu
from jax.experimental.pallas import tpu_sc as plsc

_NUM_FIELDS = 26
_PER_FIELD_VOCAB = 100000
_EMBED_DIM = 32
_BATCH = 16384

_NC = 2
_NS = 16
_L = 16
_NW = _NC * _NS
_BW = _BATCH // _NW
_NGRP = 325000
_CH = 64

_mesh = plsc.VectorSubcoreMesh(core_axis_name="c", subcore_axis_name="s")


@functools.partial(
    pl.kernel,
    out_type=jax.ShapeDtypeStruct((_NUM_FIELDS, _EMBED_DIM, _BATCH), jnp.float32),
    mesh=_mesh,
    compiler_params=pltpu.CompilerParams(needs_layout_passes=False),
    scratch_types=[
        pltpu.VMEM((_BW,), jnp.int32),
        pltpu.VMEM((_CH,), jnp.int32),
        pltpu.VMEM((_CH, 8, _EMBED_DIM), jnp.float32),
        pltpu.VMEM((_EMBED_DIM, _BW), jnp.float32),
        pltpu.SemaphoreType.DMA,
    ],
)
def _emb_lookup(xT_hbm, tg_hbm, out_hbm, xb_v, s_v, rows_v, out_v, sem):
    wid = lax.axis_index("s") * _NC + lax.axis_index("c")
    b0 = pl.multiple_of(wid * _BW, 128)

    def field_body(f, carry):
        off = f * _PER_FIELD_VOCAB
        pltpu.sync_copy(xT_hbm.at[f, pl.ds(b0, _BW)], xb_v)

        def chunk_body(q, c):
            copies = []
            for g in range(_CH // _L):
                vv = xb_v[pl.ds(q * _CH + g * _L, _L)] + off
                s_v[pl.ds(g * _L, _L)] = jnp.bitwise_and(vv, 7)
                for j in range(_L):
                    grp = lax.shift_right_logical(vv[j], 3)
                    copies.append(
                        pltpu.async_copy(
                            tg_hbm.at[pl.ds(grp, 1)],
                            rows_v.at[pl.ds(g * _L + j, 1)],
                            sem,
                        )
                    )
            for cp in copies:
                cp.wait()

            def ext_grp(g, c2):
                jvec = g * _L + lax.iota(jnp.int32, _L)
                svec = s_v[pl.ds(g * _L, _L)]
                for e in range(_EMBED_DIM):
                    evec = jnp.full((_L,), e, jnp.int32)
                    out_v[e, pl.ds(q * _CH + g * _L, _L)] = plsc.load_gather(
                        rows_v, [jvec, svec, evec]
                    )
                return c2

            lax.fori_loop(0, _CH // _L, ext_grp, 0)
            return c

        lax.fori_loop(0, _BW // _CH, chunk_body, 0)
        pltpu.sync_copy(out_v, out_hbm.at[f, :, pl.ds(b0, _BW)])
        return carry

    lax.fori_loop(0, _NUM_FIELDS, field_body, 0)


@jax.jit
def kernel(x, embedding_table):
    out = _emb_lookup(x.T, embedding_table.reshape(_NGRP, 8, _EMBED_DIM))
    return jnp.transpose(out, (2, 0, 1))
